# Initial kernel scaffold; baseline (speedup 1.0000x reference)
import jax
import jax.numpy as jnp
from jax.experimental import pallas as pl


def kernel(users, items, langs, skills, timestamps, targets, mask, emb_table, W1, b1, W2, b2, Ws, bs, Wlin, blin):
    raise NotImplementedError("write your pallas kernel here")


# speedup vs baseline: 4.0817x; 4.0817x over previous
import jax
import jax.numpy as jnp
from jax import lax
from jax.experimental import pallas as pl
from jax.experimental.pallas import tpu as pltpu
from jax.experimental.pallas import tpu_sc as plsc

L = 50
K = 2000
E = 64
B = 128
PAIRS = L * L
DP = 128
CHUNK = 40
NCHUNK = 5
H1 = 256
H2 = 128


def _sc_gather_body(table_hbm, idx_hbm, out_hbm, idx_v, rows_v, sem):
    wid = lax.axis_index("s") * 2 + lax.axis_index("c")
    base = wid * (CHUNK * NCHUNK)
    for c in range(NCHUNK):
        off = base + c * CHUNK
        pltpu.sync_copy(idx_hbm.at[pl.ds(off, CHUNK)], idx_v)
        pltpu.async_copy(table_hbm.at[idx_v], rows_v, sem).wait()
        pltpu.sync_copy(rows_v, out_hbm.at[pl.ds(off, CHUNK)])


def _sc_gather(table, idx):
    mesh = plsc.VectorSubcoreMesh(core_axis_name="c", subcore_axis_name="s")
    f = pl.kernel(
        _sc_gather_body,
        mesh=mesh,
        out_type=jax.ShapeDtypeStruct((B * L, DP), jnp.float32),
        scratch_types=[
            pltpu.VMEM((CHUNK,), jnp.int32),
            pltpu.VMEM((CHUNK, DP), jnp.float32),
            pltpu.SemaphoreType.DMA,
        ],
    )
    return f(table, idx)


def _tc_body(se_ref, ts_ref, sk_ref, tg_ref, mk_ref, us_ref, it_ref, lg_ref,
             wk_ref, wt1_ref, wt2_ref,
             w1a_ref, w1b_ref, c8_ref, b1_ref, w2t_ref, b2_ref, wsc_ref,
             bs_ref, wu_ref, wi_ref, wl_ref, blin_ref,
             loss_ref, sig_ref, lab_ref):
    f32 = jnp.float32
    se = se_ref[0]
    p = jnp.dot(se, w1a_ref[...], preferred_element_type=f32)
    q = jnp.dot(se, w1b_ref[...], preferred_element_type=f32)
    tcol = ts_ref[0]
    skcol = sk_ref[0]
    tgcol = tg_ref[0]
    padcol = (skcol == 0.0).astype(f32)

    rowi = lax.broadcasted_iota(jnp.int32, (PAIRS, L), 0) // L
    colc = lax.broadcasted_iota(jnp.int32, (PAIRS, L), 1)
    fsel = (rowi == colc).astype(f32)
    p_exp = jnp.dot(fsel, p, preferred_element_type=f32)
    q_exp = jnp.concatenate([q] * L, axis=0)
    ti = jnp.dot(fsel, tcol, preferred_element_type=f32)
    padi = jnp.dot(fsel, padcol, preferred_element_type=f32)
    tj = jnp.concatenate([tcol] * L, axis=0)
    padj = jnp.concatenate([padcol] * L, axis=0)
    tgj = jnp.concatenate([tgcol] * L, axis=0)

    dt = ti - tj
    cat = (1.0 + (dt > 1.0).astype(f32) + (dt > 3600.0).astype(f32)
           + (dt > 86400.0).astype(f32) + (dt > 604800.0).astype(f32))
    cat = jnp.where(padi + padj > 0.0, 0.0, cat)
    oh = (cat.astype(jnp.int32)
          == lax.broadcasted_iota(jnp.int32, (PAIRS, 8), 1)).astype(f32)
    dtc = jnp.dot(oh, c8_ref[...], preferred_element_type=f32)

    h1 = jnp.maximum(p_exp + q_exp + dtc + b1_ref[...], 0.0)
    h2 = jnp.maximum(
        jnp.dot(h1, w2t_ref[...], preferred_element_type=f32)
        + b2_ref[...], 0.0)
    s = jnp.tanh(
        jnp.dot(h2, wsc_ref[...], preferred_element_type=f32)
        + bs_ref[0, 0])

    ii2 = lax.broadcasted_iota(jnp.int32, (L, PAIRS), 0)
    pp2 = lax.broadcasted_iota(jnp.int32, (L, PAIRS), 1)
    tri = ((pp2 // L == ii2) & (pp2 % L < ii2)).astype(f32)
    padf = 1.0 - padcol
    vals1 = jnp.dot(tri, s, preferred_element_type=f32) * padf
    vals2 = jnp.dot(tri, s * tgj, preferred_element_type=f32) * padf

    udot = jnp.sum(us_ref[0] * wu_ref[...])
    itdot = jnp.sum(it_ref[0] * wi_ref[...], axis=1, keepdims=True)
    ldot = jnp.sum(lg_ref[0] * wl_ref[...], axis=1, keepdims=True)
    logits = (udot + itdot + ldot + wk_ref[0]
              + wt1_ref[0] * vals1 + wt2_ref[0] * vals2 + blin_ref[0, 0])
    m = mk_ref[0]
    preds = logits * m
    labels = tgcol * m
    loss_ref[0] = (jnp.maximum(preds, 0.0) - preds * labels
                   + jnp.log1p(jnp.exp(-jnp.abs(preds))))
    sig_ref[0] = 1.0 / (1.0 + jnp.exp(-preds))
    lab_ref[0] = labels


def _tc_call(interpret, se3, ts3, sk3, tg3, mk3, us3, it3, lg3, wkg, wt1g,
             wt2g, w1a, w1b, c8, b1r, w2t, b2r, wsc, bsr, wu, wi, wl, blinr):
    def perb(shape):
        return pl.BlockSpec((1,) + shape[1:], lambda i: (i, 0, 0))

    def const(arr):
        return pl.BlockSpec(arr.shape, lambda i: (0,) * arr.ndim)

    in_specs = [perb(se3.shape), perb(ts3.shape), perb(sk3.shape),
                perb(tg3.shape), perb(mk3.shape), perb(us3.shape),
                perb(it3.shape), perb(lg3.shape), perb(wkg.shape),
                perb(wt1g.shape), perb(wt2g.shape),
                const(w1a), const(w1b), const(c8), const(b1r), const(w2t),
                const(b2r), const(wsc), const(bsr), const(wu), const(wi),
                const(wl), const(blinr)]
    out_specs = [perb((B, L, 1))] * 3
    out_shape = [jax.ShapeDtypeStruct((B, L, 1), jnp.float32)] * 3
    return pl.pallas_call(
        _tc_body,
        grid=(B,),
        in_specs=in_specs,
        out_specs=out_specs,
        out_shape=out_shape,
        interpret=interpret,
    )(se3, ts3, sk3, tg3, mk3, us3, it3, lg3, wkg, wt1g, wt2g,
      w1a, w1b, c8, b1r, w2t, b2r, wsc, bsr, wu, wi, wl, blinr)


def _prep(users, items, langs, skills, timestamps, targets, mask, W1, b1,
          W2, b2, Ws, bs, blin, g):
    se3 = g[:, :E].reshape(B, L, E)
    wkg = g[:, E].reshape(B, L, 1)
    wt1g = g[:, E + 1].reshape(B, L, 1)
    wt2g = g[:, E + 2].reshape(B, L, 1)
    w1a = W1[:, :E].T
    w1b = W1[:, E:2 * E].T
    c8 = jnp.concatenate(
        [W1[:, 2 * E:2 * E + 6].T, jnp.zeros((2, H1), jnp.float32)], axis=0)
    w2t = W2.T
    wsc = Ws.T
    b1r = b1.reshape(1, H1)
    b2r = b2.reshape(1, H2)
    bsr = bs.reshape(1, 1)
    ts3 = timestamps.astype(jnp.float32).reshape(B, L, 1)
    sk3 = skills.astype(jnp.float32).reshape(B, L, 1)
    tg3 = targets.reshape(B, L, 1)
    mk3 = jnp.asarray(mask).astype(jnp.float32).reshape(B, L, 1)
    us3 = users.reshape(B, 1, 32)
    it3 = items.reshape(B, L, 32)
    lg3 = langs.reshape(B, L, 16)
    return (se3, ts3, sk3, tg3, mk3, us3, it3, lg3, wkg, wt1g, wt2g,
            w1a, w1b, c8, b1r, w2t, b2r, wsc, bsr)


def _wlin_split(Wlin):
    w = Wlin[0]
    wu = w[:32].reshape(1, 32)
    wi = w[32:64].reshape(1, 32)
    wl = w[64:80].reshape(1, 16)
    wk = w[80:80 + K]
    wt1 = w[80 + K:80 + 2 * K]
    wt2 = w[80 + 2 * K:80 + 3 * K]
    return wu, wi, wl, wk, wt1, wt2


def kernel(users, items, langs, skills, timestamps, targets, mask, emb_table,
           W1, b1, W2, b2, Ws, bs, Wlin, blin):
    wu, wi, wl, wk, wt1, wt2 = _wlin_split(Wlin)
    table = jnp.concatenate(
        [emb_table, wk[:, None], wt1[:, None], wt2[:, None],
         jnp.zeros((K, DP - E - 3), jnp.float32)], axis=1)
    idx = skills.reshape(-1).astype(jnp.int32)
    g = _sc_gather(table, idx)
    pre = _prep(users, items, langs, skills, timestamps, targets, mask,
                W1, b1, W2, b2, Ws, bs, blin, g)
    blinr = blin.reshape(1, 1)
    loss3, sig3, lab3 = _tc_call(False, *pre, wu, wi, wl, blinr)
    return (loss3.reshape(-1), sig3.reshape(-1), lab3.reshape(-1))


# speedup vs baseline: 6.6957x; 1.6404x over previous
import jax
import jax.numpy as jnp
import numpy as np
from jax import lax
from jax.experimental import pallas as pl
from jax.experimental.pallas import tpu as pltpu
from jax.experimental.pallas import tpu_sc as plsc

L = 50
K = 2000
E = 64
B = 128
SLOTS = 25 * L
DP = 128
CHUNK = 40
NCHUNK = 5
H1 = 256
H2 = 128


def _pair_tables():
    r = np.arange(25)[:, None]
    c = np.arange(L)[None, :]
    first = c < r + 1
    iof = np.where(first, r + 1, 49 - r).reshape(-1)
    jof = np.where(first, c, c - (r + 1)).reshape(-1)
    valid = ((r < 24) | (c < 25)).reshape(-1)
    slots = np.arange(SLOTS)
    fe = np.zeros((SLOTS, 2 * L), np.float32)
    fe[slots[valid], iof[valid]] = 1.0
    fe[slots[valid], L + jof[valid]] = 1.0
    tri = np.zeros((L, SLOTS), np.float32)
    tri[iof[valid], slots[valid]] = 1.0
    return fe, tri


_FE, _TRI = _pair_tables()


def _sc_gather_body(table_hbm, idx_hbm, out_hbm, idx_v, rows_v, sem):
    wid = lax.axis_index("s") * 2 + lax.axis_index("c")
    base = wid * (CHUNK * NCHUNK)
    for c in range(NCHUNK):
        off = base + c * CHUNK
        pltpu.sync_copy(idx_hbm.at[pl.ds(off, CHUNK)], idx_v)
        pltpu.async_copy(table_hbm.at[idx_v], rows_v, sem).wait()
        pltpu.sync_copy(rows_v, out_hbm.at[pl.ds(off, CHUNK)])


def _sc_gather(table, idx):
    mesh = plsc.VectorSubcoreMesh(core_axis_name="c", subcore_axis_name="s")
    f = pl.kernel(
        _sc_gather_body,
        mesh=mesh,
        out_type=jax.ShapeDtypeStruct((B * L, DP), jnp.float32),
        scratch_types=[
            pltpu.VMEM((CHUNK,), jnp.int32),
            pltpu.VMEM((CHUNK, DP), jnp.float32),
            pltpu.SemaphoreType.DMA,
        ],
    )
    return f(table, idx)


def _tc_body(se_ref, ts_ref, sk_ref, tg_ref, mk_ref, us_ref, it_ref, lg_ref,
             wk_ref, wt1_ref, wt2_ref,
             w1a_ref, w1b_ref, c8_ref, b1_ref, w2t_ref, b2_ref, wsc_ref,
             bs_ref, wu_ref, wi_ref, wl_ref, blin_ref, fe_ref, tri_ref,
             loss_ref, sig_ref, lab_ref):
    f32 = jnp.float32
    se = se_ref[0]
    p = jnp.dot(se, w1a_ref[...], preferred_element_type=f32)
    q = jnp.dot(se, w1b_ref[...], preferred_element_type=f32)
    tcol = ts_ref[0]
    skcol = sk_ref[0]
    tgcol = tg_ref[0]
    padcol = (skcol == 0.0).astype(f32)

    pq = jnp.concatenate([p, q], axis=0)
    zcol = jnp.zeros_like(tcol)
    m3 = jnp.concatenate(
        [jnp.concatenate([tcol, padcol, zcol], axis=1),
         jnp.concatenate([-tcol, padcol, tgcol], axis=1)], axis=0)
    pq_exp = jnp.dot(fe_ref[...], pq, preferred_element_type=f32)
    e3 = jnp.dot(fe_ref[...], m3, preferred_element_type=f32)
    dt = e3[:, 0:1]
    padsum = e3[:, 1:2]
    tgj = e3[:, 2:3]

    cat = (1.0 + (dt > 1.0).astype(f32) + (dt > 3600.0).astype(f32)
           + (dt > 86400.0).astype(f32) + (dt > 604800.0).astype(f32))
    cat = jnp.where(padsum > 0.0, 0.0, cat)
    oh = (cat.astype(jnp.int32)
          == lax.broadcasted_iota(jnp.int32, (SLOTS, 8), 1)).astype(f32)
    dtc = jnp.dot(oh, c8_ref[...], preferred_element_type=f32)

    h1 = jnp.maximum(pq_exp + dtc + b1_ref[...], 0.0)
    h2 = jnp.maximum(
        jnp.dot(h1, w2t_ref[...], preferred_element_type=f32)
        + b2_ref[...], 0.0)
    s = jnp.tanh(
        jnp.dot(h2, wsc_ref[...], preferred_element_type=f32)
        + bs_ref[0, 0])

    padf = 1.0 - padcol
    vals1 = jnp.dot(tri_ref[...], s, preferred_element_type=f32) * padf
    vals2 = jnp.dot(tri_ref[...], s * tgj,
                    preferred_element_type=f32) * padf

    udot = jnp.sum(us_ref[0] * wu_ref[...])
    itdot = jnp.sum(it_ref[0] * wi_ref[...], axis=1, keepdims=True)
    ldot = jnp.sum(lg_ref[0] * wl_ref[...], axis=1, keepdims=True)
    logits = (udot + itdot + ldot + wk_ref[0]
              + wt1_ref[0] * vals1 + wt2_ref[0] * vals2 + blin_ref[0, 0])
    m = mk_ref[0]
    preds = logits * m
    labels = tgcol * m
    loss_ref[0] = (jnp.maximum(preds, 0.0) - preds * labels
                   + jnp.log1p(jnp.exp(-jnp.abs(preds))))
    sig_ref[0] = 1.0 / (1.0 + jnp.exp(-preds))
    lab_ref[0] = labels


def _tc_call(interpret, se3, ts3, sk3, tg3, mk3, us3, it3, lg3, wkg, wt1g,
             wt2g, w1a, w1b, c8, b1r, w2t, b2r, wsc, bsr, wu, wi, wl, blinr,
             fe, tri):
    def perb(shape):
        return pl.BlockSpec((1,) + shape[1:], lambda i: (i, 0, 0))

    def const(arr):
        return pl.BlockSpec(arr.shape, lambda i: (0,) * arr.ndim)

    in_specs = [perb(se3.shape), perb(ts3.shape), perb(sk3.shape),
                perb(tg3.shape), perb(mk3.shape), perb(us3.shape),
                perb(it3.shape), perb(lg3.shape), perb(wkg.shape),
                perb(wt1g.shape), perb(wt2g.shape),
                const(w1a), const(w1b), const(c8), const(b1r), const(w2t),
                const(b2r), const(wsc), const(bsr), const(wu), const(wi),
                const(wl), const(blinr), const(fe), const(tri)]
    out_specs = [perb((B, L, 1))] * 3
    out_shape = [jax.ShapeDtypeStruct((B, L, 1), jnp.float32)] * 3
    return pl.pallas_call(
        _tc_body,
        grid=(B,),
        in_specs=in_specs,
        out_specs=out_specs,
        out_shape=out_shape,
        interpret=interpret,
    )(se3, ts3, sk3, tg3, mk3, us3, it3, lg3, wkg, wt1g, wt2g,
      w1a, w1b, c8, b1r, w2t, b2r, wsc, bsr, wu, wi, wl, blinr, fe, tri)


def _prep(users, items, langs, skills, timestamps, targets, mask, W1, b1,
          W2, b2, Ws, bs, blin, g):
    se3 = g[:, :E].reshape(B, L, E)
    wkg = g[:, E].reshape(B, L, 1)
    wt1g = g[:, E + 1].reshape(B, L, 1)
    wt2g = g[:, E + 2].reshape(B, L, 1)
    w1a = W1[:, :E].T
    w1b = W1[:, E:2 * E].T
    c8 = jnp.concatenate(
        [W1[:, 2 * E:2 * E + 6].T, jnp.zeros((2, H1), jnp.float32)], axis=0)
    w2t = W2.T
    wsc = Ws.T
    b1r = b1.reshape(1, H1)
    b2r = b2.reshape(1, H2)
    bsr = bs.reshape(1, 1)
    ts3 = timestamps.astype(jnp.float32).reshape(B, L, 1)
    sk3 = skills.astype(jnp.float32).reshape(B, L, 1)
    tg3 = targets.reshape(B, L, 1)
    mk3 = jnp.asarray(mask).astype(jnp.float32).reshape(B, L, 1)
    us3 = users.reshape(B, 1, 32)
    it3 = items.reshape(B, L, 32)
    lg3 = langs.reshape(B, L, 16)
    return (se3, ts3, sk3, tg3, mk3, us3, it3, lg3, wkg, wt1g, wt2g,
            w1a, w1b, c8, b1r, w2t, b2r, wsc, bsr)


def _wlin_split(Wlin):
    w = Wlin[0]
    wu = w[:32].reshape(1, 32)
    wi = w[32:64].reshape(1, 32)
    wl = w[64:80].reshape(1, 16)
    wk = w[80:80 + K]
    wt1 = w[80 + K:80 + 2 * K]
    wt2 = w[80 + 2 * K:80 + 3 * K]
    return wu, wi, wl, wk, wt1, wt2


def kernel(users, items, langs, skills, timestamps, targets, mask, emb_table,
           W1, b1, W2, b2, Ws, bs, Wlin, blin):
    wu, wi, wl, wk, wt1, wt2 = _wlin_split(Wlin)
    table = jnp.concatenate(
        [emb_table, wk[:, None], wt1[:, None], wt2[:, None],
         jnp.zeros((K, DP - E - 3), jnp.float32)], axis=1)
    idx = skills.reshape(-1).astype(jnp.int32)
    g = _sc_gather(table, idx)
    pre = _prep(users, items, langs, skills, timestamps, targets, mask,
                W1, b1, W2, b2, Ws, bs, blin, g)
    blinr = blin.reshape(1, 1)
    loss3, sig3, lab3 = _tc_call(False, *pre, wu, wi, wl, blinr,
                                 jnp.asarray(_FE), jnp.asarray(_TRI))
    return (loss3.reshape(-1), sig3.reshape(-1), lab3.reshape(-1))


# speedup vs baseline: 7.6159x; 1.1374x over previous
import jax
import jax.numpy as jnp
import numpy as np
from jax import lax
from jax.experimental import pallas as pl
from jax.experimental.pallas import tpu as pltpu
from jax.experimental.pallas import tpu_sc as plsc

L = 50
K = 2000
E = 64
B = 128
SLOTS = 25 * L
DP = 128
CHUNK = 40
NCHUNK = 5
H1 = 256
H2 = 128


def _pair_tables():
    r = np.arange(25)[:, None]
    c = np.arange(L)[None, :]
    first = c < r + 1
    iof = np.where(first, r + 1, 49 - r).reshape(-1)
    jof = np.where(first, c, c - (r + 1)).reshape(-1)
    valid = ((r < 24) | (c < 25)).reshape(-1)
    slots = np.arange(SLOTS)
    fe = np.zeros((SLOTS, 2 * L), np.float32)
    fe[slots[valid], iof[valid]] = 1.0
    fe[slots[valid], L + jof[valid]] = 1.0
    tri = np.zeros((L, SLOTS), np.float32)
    tri[iof[valid], slots[valid]] = 1.0
    return fe, tri


_FE, _TRI = _pair_tables()


def _sc_gather_body(table_hbm, idx_hbm, out_hbm, idx_v, rows_v, sem):
    wid = lax.axis_index("s") * 2 + lax.axis_index("c")
    base = wid * (CHUNK * NCHUNK)
    for c in range(NCHUNK):
        off = base + c * CHUNK
        pltpu.sync_copy(idx_hbm.at[pl.ds(off, CHUNK)], idx_v)
        pltpu.async_copy(table_hbm.at[idx_v], rows_v, sem).wait()
        pltpu.sync_copy(rows_v, out_hbm.at[pl.ds(off, CHUNK)])


def _sc_gather(table, idx):
    mesh = plsc.VectorSubcoreMesh(core_axis_name="c", subcore_axis_name="s")
    f = pl.kernel(
        _sc_gather_body,
        mesh=mesh,
        out_type=jax.ShapeDtypeStruct((B * L, DP), jnp.float32),
        scratch_types=[
            pltpu.VMEM((CHUNK,), jnp.int32),
            pltpu.VMEM((CHUNK, DP), jnp.float32),
            pltpu.SemaphoreType.DMA,
        ],
    )
    return f(table, idx)


def _tc_body(se_ref, ts_ref, sk_ref, tg_ref, mk_ref, us_ref, it_ref, lg_ref,
             wk_ref, wt1_ref, wt2_ref,
             w1a_ref, w1b_ref, c8_ref, w2t_ref, b2_ref, wsc_ref,
             bs_ref, wu_ref, wi_ref, wl_ref, blin_ref, fe_ref, tri_ref,
             febf_ref, loss_ref, sig_ref, lab_ref):
    f32 = jnp.float32
    se = se_ref[0]
    p = jnp.dot(se, w1a_ref[...], preferred_element_type=f32)
    q = jnp.dot(se, w1b_ref[...], preferred_element_type=f32)
    tcol = ts_ref[0]
    skcol = sk_ref[0]
    tgcol = tg_ref[0]
    padcol = (skcol == 0.0).astype(f32)

    pq = jnp.concatenate([p, q], axis=0)
    zcol = jnp.zeros_like(tcol)
    m3 = jnp.concatenate(
        [jnp.concatenate([tcol, padcol, zcol], axis=1),
         jnp.concatenate([-tcol, padcol, tgcol], axis=1)], axis=0)
    e3 = jnp.dot(fe_ref[...], m3, preferred_element_type=f32)
    dt = e3[:, 0:1]
    padsum = e3[:, 1:2]
    tgj = e3[:, 2:3]

    cat = (1.0 + (dt > 1.0).astype(f32) + (dt > 3600.0).astype(f32)
           + (dt > 86400.0).astype(f32) + (dt > 604800.0).astype(f32))
    cat = jnp.where(padsum > 0.0, 0.0, cat)
    oh = (cat.astype(jnp.int32)
          == lax.broadcasted_iota(jnp.int32, (SLOTS, 8), 1))

    bf = jnp.bfloat16
    lhs = jnp.concatenate([febf_ref[...], oh.astype(bf)], axis=1)
    rhs = jnp.concatenate([pq.astype(bf), c8_ref[...]], axis=0)
    h1 = jnp.maximum(jnp.dot(lhs, rhs, preferred_element_type=f32), 0.0)
    h2 = jnp.maximum(
        jnp.dot(h1.astype(bf), w2t_ref[...], preferred_element_type=f32)
        + b2_ref[...], 0.0)
    s = jnp.tanh(
        jnp.dot(h2, wsc_ref[...], preferred_element_type=f32)
        + bs_ref[0, 0])

    padf = 1.0 - padcol
    vals1 = jnp.dot(tri_ref[...], s, preferred_element_type=f32) * padf
    vals2 = jnp.dot(tri_ref[...], s * tgj,
                    preferred_element_type=f32) * padf

    udot = jnp.sum(us_ref[0] * wu_ref[...])
    itdot = jnp.sum(it_ref[0] * wi_ref[...], axis=1, keepdims=True)
    ldot = jnp.sum(lg_ref[0] * wl_ref[...], axis=1, keepdims=True)
    logits = (udot + itdot + ldot + wk_ref[0]
              + wt1_ref[0] * vals1 + wt2_ref[0] * vals2 + blin_ref[0, 0])
    m = mk_ref[0]
    preds = logits * m
    labels = tgcol * m
    loss_ref[0] = (jnp.maximum(preds, 0.0) - preds * labels
                   + jnp.log1p(jnp.exp(-jnp.abs(preds))))
    sig_ref[0] = 1.0 / (1.0 + jnp.exp(-preds))
    lab_ref[0] = labels


def _tc_call(interpret, se3, ts3, sk3, tg3, mk3, us3, it3, lg3, wkg, wt1g,
             wt2g, w1a, w1b, c8, w2t, b2r, wsc, bsr, wu, wi, wl, blinr,
             fe, tri, febf):
    def perb(shape):
        return pl.BlockSpec((1,) + shape[1:], lambda i: (i, 0, 0))

    def const(arr):
        return pl.BlockSpec(arr.shape, lambda i: (0,) * arr.ndim)

    in_specs = [perb(se3.shape), perb(ts3.shape), perb(sk3.shape),
                perb(tg3.shape), perb(mk3.shape), perb(us3.shape),
                perb(it3.shape), perb(lg3.shape), perb(wkg.shape),
                perb(wt1g.shape), perb(wt2g.shape),
                const(w1a), const(w1b), const(c8), const(w2t),
                const(b2r), const(wsc), const(bsr), const(wu), const(wi),
                const(wl), const(blinr), const(fe), const(tri), const(febf)]
    out_specs = [perb((B, L, 1))] * 3
    out_shape = [jax.ShapeDtypeStruct((B, L, 1), jnp.float32)] * 3
    return pl.pallas_call(
        _tc_body,
        grid=(B,),
        in_specs=in_specs,
        out_specs=out_specs,
        out_shape=out_shape,
        interpret=interpret,
    )(se3, ts3, sk3, tg3, mk3, us3, it3, lg3, wkg, wt1g, wt2g,
      w1a, w1b, c8, w2t, b2r, wsc, bsr, wu, wi, wl, blinr, fe, tri, febf)


def _prep(users, items, langs, skills, timestamps, targets, mask, W1, b1,
          W2, b2, Ws, bs, blin, g):
    se3 = g[:, :E].reshape(B, L, E)
    wkg = g[:, E].reshape(B, L, 1)
    wt1g = g[:, E + 1].reshape(B, L, 1)
    wt2g = g[:, E + 2].reshape(B, L, 1)
    w1a = W1[:, :E].T
    w1b = W1[:, E:2 * E].T
    c8 = (jnp.concatenate(
        [W1[:, 2 * E:2 * E + 6].T, jnp.zeros((2, H1), jnp.float32)], axis=0)
        + b1[None, :]).astype(jnp.bfloat16)
    w2t = W2.T.astype(jnp.bfloat16)
    wsc = Ws.T
    b2r = b2.reshape(1, H2)
    bsr = bs.reshape(1, 1)
    ts3 = timestamps.astype(jnp.float32).reshape(B, L, 1)
    sk3 = skills.astype(jnp.float32).reshape(B, L, 1)
    tg3 = targets.reshape(B, L, 1)
    mk3 = jnp.asarray(mask).astype(jnp.float32).reshape(B, L, 1)
    us3 = users.reshape(B, 1, 32)
    it3 = items.reshape(B, L, 32)
    lg3 = langs.reshape(B, L, 16)
    return (se3, ts3, sk3, tg3, mk3, us3, it3, lg3, wkg, wt1g, wt2g,
            w1a, w1b, c8, w2t, b2r, wsc, bsr)


def _wlin_split(Wlin):
    w = Wlin[0]
    wu = w[:32].reshape(1, 32)
    wi = w[32:64].reshape(1, 32)
    wl = w[64:80].reshape(1, 16)
    wk = w[80:80 + K]
    wt1 = w[80 + K:80 + 2 * K]
    wt2 = w[80 + 2 * K:80 + 3 * K]
    return wu, wi, wl, wk, wt1, wt2


def kernel(users, items, langs, skills, timestamps, targets, mask, emb_table,
           W1, b1, W2, b2, Ws, bs, Wlin, blin):
    wu, wi, wl, wk, wt1, wt2 = _wlin_split(Wlin)
    table = jnp.concatenate(
        [emb_table, wk[:, None], wt1[:, None], wt2[:, None],
         jnp.zeros((K, DP - E - 3), jnp.float32)], axis=1)
    idx = skills.reshape(-1).astype(jnp.int32)
    g = _sc_gather(table, idx)
    pre = _prep(users, items, langs, skills, timestamps, targets, mask,
                W1, b1, W2, b2, Ws, bs, blin, g)
    blinr = blin.reshape(1, 1)
    loss3, sig3, lab3 = _tc_call(False, *pre, wu, wi, wl, blinr,
                                 jnp.asarray(_FE), jnp.asarray(_TRI),
                                 jnp.asarray(_FE).astype(jnp.bfloat16))
    return (loss3.reshape(-1), sig3.reshape(-1), lab3.reshape(-1))


# speedup vs baseline: 10.3493x; 1.3589x over previous
import jax
import jax.numpy as jnp
import numpy as np
from jax import lax
from jax.experimental import pallas as pl
from jax.experimental.pallas import tpu as pltpu
from jax.experimental.pallas import tpu_sc as plsc

L = 50
K = 2000
E = 64
B = 128
SLOTS = 25 * L
DP = 128
CHUNK = 40
NCHUNK = 5
H1 = 256
H2 = 128


def _pair_tables():
    r = np.arange(25)[:, None]
    c = np.arange(L)[None, :]
    first = c < r + 1
    iof = np.where(first, r + 1, 49 - r).reshape(-1)
    jof = np.where(first, c, c - (r + 1)).reshape(-1)
    valid = ((r < 24) | (c < 25)).reshape(-1)
    slots = np.arange(SLOTS)
    fe = np.zeros((SLOTS, 2 * L), np.float32)
    fe[slots[valid], iof[valid]] = 1.0
    fe[slots[valid], L + jof[valid]] = 1.0
    tri = np.zeros((L, SLOTS), np.float32)
    tri[iof[valid], slots[valid]] = 1.0
    return fe, tri


_FE, _TRI = _pair_tables()


def _sc_gather_body(table_hbm, idx_hbm, out_hbm, idx_v, rows_v, sem):
    wid = lax.axis_index("s") * 2 + lax.axis_index("c")
    base = wid * (CHUNK * NCHUNK)
    for c in range(NCHUNK):
        off = base + c * CHUNK
        pltpu.sync_copy(idx_hbm.at[pl.ds(off, CHUNK)], idx_v)
        pltpu.async_copy(table_hbm.at[idx_v], rows_v, sem).wait()
        pltpu.sync_copy(rows_v, out_hbm.at[pl.ds(off, CHUNK)])


def _sc_gather(table, idx):
    mesh = plsc.VectorSubcoreMesh(core_axis_name="c", subcore_axis_name="s")
    f = pl.kernel(
        _sc_gather_body,
        mesh=mesh,
        out_type=jax.ShapeDtypeStruct((B * L, DP), jnp.float32),
        scratch_types=[
            pltpu.VMEM((CHUNK,), jnp.int32),
            pltpu.VMEM((CHUNK, DP), jnp.float32),
            pltpu.SemaphoreType.DMA,
        ],
    )
    return f(table, idx)


def _tc_body(se_ref, ts_ref, sk_ref, tg_ref, mk_ref, us_ref, it_ref, lg_ref,
             wk_ref, wt1_ref, wt2_ref,
             w1a_ref, w1b_ref, c8_ref, w2_ref, b2_ref, wsc_ref,
             bs_ref, wu_ref, wi_ref, wl_ref, blin_ref, fe_ref, tri_ref,
             febf_ref, loss_ref, sig_ref, lab_ref):
    f32 = jnp.float32
    bf = jnp.bfloat16
    set_ = se_ref[0]
    pt = jnp.dot(w1a_ref[...], set_, preferred_element_type=f32)
    qt = jnp.dot(w1b_ref[...], set_, preferred_element_type=f32)
    trow = ts_ref[0]
    skrow = sk_ref[0]
    tgrow = tg_ref[0]
    padrow = (skrow == 0.0).astype(f32)

    zrow = jnp.zeros_like(trow)
    m3 = jnp.concatenate(
        [jnp.concatenate([trow, -trow], axis=1),
         jnp.concatenate([padrow, padrow], axis=1),
         jnp.concatenate([zrow, tgrow], axis=1)], axis=0)
    e3 = jnp.dot(m3, fe_ref[...], preferred_element_type=f32)
    dt = e3[0:1, :]
    padsum = e3[1:2, :]
    tgj = e3[2:3, :]

    cat = (1.0 + (dt > 1.0).astype(f32) + (dt > 3600.0).astype(f32)
           + (dt > 86400.0).astype(f32) + (dt > 604800.0).astype(f32))
    cat = jnp.where(padsum > 0.0, 0.0, cat)
    oh = (cat.astype(jnp.int32)
          == lax.broadcasted_iota(jnp.int32, (8, SLOTS), 0))

    aall = jnp.concatenate([pt, qt, c8_ref[...]],
                           axis=1).astype(bf)
    lhs = jnp.concatenate([febf_ref[...], oh.astype(bf)], axis=0)
    h1 = jnp.maximum(jnp.dot(aall, lhs, preferred_element_type=f32), 0.0)
    h2 = jnp.maximum(
        jnp.dot(w2_ref[...], h1.astype(bf), preferred_element_type=f32)
        + b2_ref[...], 0.0)
    s = jnp.tanh(
        jnp.dot(wsc_ref[...], h2, preferred_element_type=f32)
        + bs_ref[0, 0])

    padf = 1.0 - padrow
    vals1 = jnp.dot(s, tri_ref[...], preferred_element_type=f32) * padf
    vals2 = jnp.dot(s * tgj, tri_ref[...],
                    preferred_element_type=f32) * padf

    udot = jnp.sum(us_ref[0] * wu_ref[...])
    itdot = jnp.dot(wi_ref[...], it_ref[0], preferred_element_type=f32)
    ldot = jnp.dot(wl_ref[...], lg_ref[0], preferred_element_type=f32)
    logits = (udot + itdot + ldot + wk_ref[0]
              + wt1_ref[0] * vals1 + wt2_ref[0] * vals2 + blin_ref[0, 0])
    m = mk_ref[0]
    preds = logits * m
    labels = tgrow * m
    loss_ref[0] = (jnp.maximum(preds, 0.0) - preds * labels
                   + jnp.log1p(jnp.exp(-jnp.abs(preds))))
    sig_ref[0] = 1.0 / (1.0 + jnp.exp(-preds))
    lab_ref[0] = labels


def _tc_call(interpret, se3, ts3, sk3, tg3, mk3, us3, it3, lg3, wkg, wt1g,
             wt2g, w1a, w1b, c8, w2t, b2r, wsc, bsr, wu, wi, wl, blinr,
             fe, tri, febf):
    def perb(shape):
        return pl.BlockSpec((1,) + shape[1:], lambda i: (i, 0, 0))

    def const(arr):
        return pl.BlockSpec(arr.shape, lambda i: (0,) * arr.ndim)

    in_specs = [perb(se3.shape), perb(ts3.shape), perb(sk3.shape),
                perb(tg3.shape), perb(mk3.shape), perb(us3.shape),
                perb(it3.shape), perb(lg3.shape), perb(wkg.shape),
                perb(wt1g.shape), perb(wt2g.shape),
                const(w1a), const(w1b), const(c8), const(w2t),
                const(b2r), const(wsc), const(bsr), const(wu), const(wi),
                const(wl), const(blinr), const(fe), const(tri), const(febf)]
    out_specs = [perb((B, 1, L))] * 3
    out_shape = [jax.ShapeDtypeStruct((B, 1, L), jnp.float32)] * 3
    return pl.pallas_call(
        _tc_body,
        grid=(B,),
        in_specs=in_specs,
        out_specs=out_specs,
        out_shape=out_shape,
        interpret=interpret,
    )(se3, ts3, sk3, tg3, mk3, us3, it3, lg3, wkg, wt1g, wt2g,
      w1a, w1b, c8, w2t, b2r, wsc, bsr, wu, wi, wl, blinr, fe, tri, febf)


def _prep(users, items, langs, skills, timestamps, targets, mask, W1, b1,
          W2, b2, Ws, bs, blin, g):
    set3 = g[:, :E].reshape(B, L, E).transpose(0, 2, 1)
    wkg = g[:, E].reshape(B, 1, L)
    wt1g = g[:, E + 1].reshape(B, 1, L)
    wt2g = g[:, E + 2].reshape(B, 1, L)
    w1a = W1[:, :E]
    w1b = W1[:, E:2 * E]
    c8 = (jnp.concatenate(
        [W1[:, 2 * E:2 * E + 6], jnp.zeros((H1, 2), jnp.float32)], axis=1)
        + b1[:, None])
    w2 = W2.astype(jnp.bfloat16)
    wsc = Ws
    b2r = b2.reshape(H2, 1)
    bsr = bs.reshape(1, 1)
    ts3 = timestamps.astype(jnp.float32).reshape(B, 1, L)
    sk3 = skills.astype(jnp.float32).reshape(B, 1, L)
    tg3 = targets.reshape(B, 1, L)
    mk3 = jnp.asarray(mask).astype(jnp.float32).reshape(B, 1, L)
    us3 = users.reshape(B, 1, 32)
    it3 = items.reshape(B, L, 32).transpose(0, 2, 1)
    lg3 = langs.reshape(B, L, 16).transpose(0, 2, 1)
    return (set3, ts3, sk3, tg3, mk3, us3, it3, lg3, wkg, wt1g, wt2g,
            w1a, w1b, c8, w2, b2r, wsc, bsr)


def _wlin_split(Wlin):
    w = Wlin[0]
    wu = w[:32].reshape(1, 32)
    wi = w[32:64].reshape(1, 32)
    wl = w[64:80].reshape(1, 16)
    wk = w[80:80 + K]
    wt1 = w[80 + K:80 + 2 * K]
    wt2 = w[80 + 2 * K:80 + 3 * K]
    return wu, wi, wl, wk, wt1, wt2


def kernel(users, items, langs, skills, timestamps, targets, mask, emb_table,
           W1, b1, W2, b2, Ws, bs, Wlin, blin):
    wu, wi, wl, wk, wt1, wt2 = _wlin_split(Wlin)
    table = jnp.concatenate(
        [emb_table, wk[:, None], wt1[:, None], wt2[:, None],
         jnp.zeros((K, DP - E - 3), jnp.float32)], axis=1)
    idx = skills.reshape(-1).astype(jnp.int32)
    g = _sc_gather(table, idx)
    pre = _prep(users, items, langs, skills, timestamps, targets, mask,
                W1, b1, W2, b2, Ws, bs, blin, g)
    blinr = blin.reshape(1, 1)
    fet = jnp.asarray(_FE.T.copy())
    trit = jnp.asarray(_TRI.T.copy())
    loss3, sig3, lab3 = _tc_call(False, *pre, wu, wi, wl, blinr,
                                 fet, trit, fet.astype(jnp.bfloat16))
    return (loss3.reshape(-1), sig3.reshape(-1), lab3.reshape(-1))


# speedup vs baseline: 11.4104x; 1.1025x over previous
import jax
import jax.numpy as jnp
import numpy as np
from jax import lax
from jax.experimental import pallas as pl
from jax.experimental.pallas import tpu as pltpu
from jax.experimental.pallas import tpu_sc as plsc

L = 50
K = 2000
E = 64
B = 128
SLOTS = 25 * L
DP = 128
CHUNK = 40
NCHUNK = 5
H1 = 256
H2 = 128
NB = 2


def _pair_tables():
    r = np.arange(25)[:, None]
    c = np.arange(L)[None, :]
    first = c < r + 1
    iof = np.where(first, r + 1, 49 - r).reshape(-1)
    jof = np.where(first, c, c - (r + 1)).reshape(-1)
    valid = ((r < 24) | (c < 25)).reshape(-1)
    slots = np.arange(SLOTS)
    fe = np.zeros((SLOTS, 2 * L), np.float32)
    fe[slots[valid], iof[valid]] = 1.0
    fe[slots[valid], L + jof[valid]] = 1.0
    tri = np.zeros((L, SLOTS), np.float32)
    tri[iof[valid], slots[valid]] = 1.0
    return fe, tri


_FE, _TRI = _pair_tables()


def _sc_gather_body(table_hbm, idx_hbm, out_hbm, idx_v, rows_v, sem):
    wid = lax.axis_index("s") * 2 + lax.axis_index("c")
    base = wid * (CHUNK * NCHUNK)
    for c in range(NCHUNK):
        off = base + c * CHUNK
        pltpu.sync_copy(idx_hbm.at[pl.ds(off, CHUNK)], idx_v)
        pltpu.async_copy(table_hbm.at[idx_v], rows_v, sem).wait()
        pltpu.sync_copy(rows_v, out_hbm.at[pl.ds(off, CHUNK)])


def _sc_gather(table, idx):
    mesh = plsc.VectorSubcoreMesh(core_axis_name="c", subcore_axis_name="s")
    f = pl.kernel(
        _sc_gather_body,
        mesh=mesh,
        out_type=jax.ShapeDtypeStruct((B * L, DP), jnp.float32),
        scratch_types=[
            pltpu.VMEM((CHUNK,), jnp.int32),
            pltpu.VMEM((CHUNK, DP), jnp.float32),
            pltpu.SemaphoreType.DMA,
        ],
    )
    return f(table, idx)


def _tc_body(se_ref, ts_ref, sk_ref, tg_ref, mk_ref, us_ref, it_ref, lg_ref,
             wk_ref, wt1_ref, wt2_ref,
             w1a_ref, w1b_ref, c8_ref, w2_ref, b2_ref, wsc_ref,
             bs_ref, wu_ref, wi_ref, wl_ref, blin_ref, fe_ref, tri_ref,
             febf_ref, loss_ref, sig_ref, lab_ref):
    f32 = jnp.float32
    bf = jnp.bfloat16

    def one(ib):
        set_ = se_ref[ib]
        pt = jnp.dot(w1a_ref[...], set_, preferred_element_type=f32)
        qt = jnp.dot(w1b_ref[...], set_, preferred_element_type=f32)
        trow = ts_ref[ib]
        skrow = sk_ref[ib]
        tgrow = tg_ref[ib]
        padrow = (skrow == 0.0).astype(f32)

        zrow = jnp.zeros_like(trow)
        m3 = jnp.concatenate(
            [jnp.concatenate([trow, -trow], axis=1),
             jnp.concatenate([padrow, padrow], axis=1),
             jnp.concatenate([zrow, tgrow], axis=1)], axis=0)
        e3 = jnp.dot(m3, fe_ref[...], preferred_element_type=f32)
        dt = e3[0:1, :]
        padsum = e3[1:2, :]
        tgj = e3[2:3, :]

        cat = (1.0 + (dt > 1.0).astype(f32) + (dt > 3600.0).astype(f32)
               + (dt > 86400.0).astype(f32) + (dt > 604800.0).astype(f32))
        cat = jnp.where(padsum > 0.0, 0.0, cat)
        oh = (cat.astype(jnp.int32)
              == lax.broadcasted_iota(jnp.int32, (8, SLOTS), 0))

        aall = jnp.concatenate([pt, qt, c8_ref[...]],
                               axis=1).astype(bf)
        lhs = jnp.concatenate([febf_ref[...], oh.astype(bf)], axis=0)
        h1 = jnp.maximum(jnp.dot(aall, lhs, preferred_element_type=f32),
                         0.0).astype(bf)
        h2 = jnp.maximum(
            jnp.dot(w2_ref[...], h1, preferred_element_type=f32)
            + b2_ref[...], 0.0)
        s = jnp.tanh(
            jnp.dot(wsc_ref[...], h2, preferred_element_type=f32)
            + bs_ref[0, 0])

        padf = 1.0 - padrow
        sv = jnp.concatenate([s, s * tgj], axis=0)
        vals = jnp.dot(sv, tri_ref[...], preferred_element_type=f32)
        vals1 = vals[0:1, :] * padf
        vals2 = vals[1:2, :] * padf

        udot = jnp.sum(us_ref[ib] * wu_ref[...])
        itdot = jnp.dot(wi_ref[...], it_ref[ib], preferred_element_type=f32)
        ldot = jnp.dot(wl_ref[...], lg_ref[ib], preferred_element_type=f32)
        logits = (udot + itdot + ldot + wk_ref[ib]
                  + wt1_ref[ib] * vals1 + wt2_ref[ib] * vals2
                  + blin_ref[0, 0])
        m = mk_ref[ib]
        preds = logits * m
        labels = tgrow * m
        loss_ref[ib] = (jnp.maximum(preds, 0.0) - preds * labels
                        + jnp.log1p(jnp.exp(-jnp.abs(preds))))
        sig_ref[ib] = 1.0 / (1.0 + jnp.exp(-preds))
        lab_ref[ib] = labels

    for ib in range(NB):
        one(ib)


def _tc_call(interpret, se3, ts3, sk3, tg3, mk3, us3, it3, lg3, wkg, wt1g,
             wt2g, w1a, w1b, c8, w2t, b2r, wsc, bsr, wu, wi, wl, blinr,
             fe, tri, febf):
    def perb(shape):
        return pl.BlockSpec((NB,) + shape[1:], lambda i: (i, 0, 0))

    def const(arr):
        return pl.BlockSpec(arr.shape, lambda i: (0,) * arr.ndim)

    in_specs = [perb(se3.shape), perb(ts3.shape), perb(sk3.shape),
                perb(tg3.shape), perb(mk3.shape), perb(us3.shape),
                perb(it3.shape), perb(lg3.shape), perb(wkg.shape),
                perb(wt1g.shape), perb(wt2g.shape),
                const(w1a), const(w1b), const(c8), const(w2t),
                const(b2r), const(wsc), const(bsr), const(wu), const(wi),
                const(wl), const(blinr), const(fe), const(tri), const(febf)]
    out_specs = [perb((B, 1, L))] * 3
    out_shape = [jax.ShapeDtypeStruct((B, 1, L), jnp.float32)] * 3
    return pl.pallas_call(
        _tc_body,
        grid=(B // NB,),
        in_specs=in_specs,
        out_specs=out_specs,
        out_shape=out_shape,
        interpret=interpret,
    )(se3, ts3, sk3, tg3, mk3, us3, it3, lg3, wkg, wt1g, wt2g,
      w1a, w1b, c8, w2t, b2r, wsc, bsr, wu, wi, wl, blinr, fe, tri, febf)


def _prep(users, items, langs, skills, timestamps, targets, mask, W1, b1,
          W2, b2, Ws, bs, blin, g):
    set3 = g[:, :E].reshape(B, L, E).transpose(0, 2, 1)
    wkg = g[:, E].reshape(B, 1, L)
    wt1g = g[:, E + 1].reshape(B, 1, L)
    wt2g = g[:, E + 2].reshape(B, 1, L)
    w1a = W1[:, :E]
    w1b = W1[:, E:2 * E]
    c8 = (jnp.concatenate(
        [W1[:, 2 * E:2 * E + 6], jnp.zeros((H1, 2), jnp.float32)], axis=1)
        + b1[:, None])
    w2 = W2.astype(jnp.bfloat16)
    wsc = Ws
    b2r = b2.reshape(H2, 1)
    bsr = bs.reshape(1, 1)
    ts3 = timestamps.astype(jnp.float32).reshape(B, 1, L)
    sk3 = skills.astype(jnp.float32).reshape(B, 1, L)
    tg3 = targets.reshape(B, 1, L)
    mk3 = jnp.asarray(mask).astype(jnp.float32).reshape(B, 1, L)
    us3 = users.reshape(B, 1, 32)
    it3 = items.reshape(B, L, 32).transpose(0, 2, 1)
    lg3 = langs.reshape(B, L, 16).transpose(0, 2, 1)
    return (set3, ts3, sk3, tg3, mk3, us3, it3, lg3, wkg, wt1g, wt2g,
            w1a, w1b, c8, w2, b2r, wsc, bsr)


def _wlin_split(Wlin):
    w = Wlin[0]
    wu = w[:32].reshape(1, 32)
    wi = w[32:64].reshape(1, 32)
    wl = w[64:80].reshape(1, 16)
    wk = w[80:80 + K]
    wt1 = w[80 + K:80 + 2 * K]
    wt2 = w[80 + 2 * K:80 + 3 * K]
    return wu, wi, wl, wk, wt1, wt2


def kernel(users, items, langs, skills, timestamps, targets, mask, emb_table,
           W1, b1, W2, b2, Ws, bs, Wlin, blin):
    wu, wi, wl, wk, wt1, wt2 = _wlin_split(Wlin)
    table = jnp.concatenate(
        [emb_table, wk[:, None], wt1[:, None], wt2[:, None],
         jnp.zeros((K, DP - E - 3), jnp.float32)], axis=1)
    idx = skills.reshape(-1).astype(jnp.int32)
    g = _sc_gather(table, idx)
    pre = _prep(users, items, langs, skills, timestamps, targets, mask,
                W1, b1, W2, b2, Ws, bs, blin, g)
    blinr = blin.reshape(1, 1)
    fet = jnp.asarray(_FE.T.copy())
    trit = jnp.asarray(_TRI.T.copy())
    loss3, sig3, lab3 = _tc_call(False, *pre, wu, wi, wl, blinr,
                                 fet, trit, fet.astype(jnp.bfloat16))
    return (loss3.reshape(-1), sig3.reshape(-1), lab3.reshape(-1))
